# Initial kernel scaffold; baseline (speedup 1.0000x reference)
import jax
import jax.numpy as jnp
from jax.experimental import pallas as pl


def kernel(x, emb_weight):
    raise NotImplementedError("write your pallas kernel here")


# speedup vs baseline: 1.7191x; 1.7191x over previous
import jax
import jax.numpy as jnp
from jax.experimental import pallas as pl

BATCH = 4
SEQ_BLK = 512


def _add_kernel(x_ref, emb_ref, out_ref):
    out_ref[...] = x_ref[...] + emb_ref[...][None, :, :]


def kernel(x, emb_weight):
    batch, seq_len, d_model = x.shape
    grid = (seq_len // SEQ_BLK,)
    return pl.pallas_call(
        _add_kernel,
        grid=grid,
        in_specs=[
            pl.BlockSpec((batch, SEQ_BLK, d_model), lambda j: (0, j, 0)),
            pl.BlockSpec((SEQ_BLK, d_model), lambda j: (j, 0)),
        ],
        out_specs=pl.BlockSpec((batch, SEQ_BLK, d_model), lambda j: (0, j, 0)),
        out_shape=jax.ShapeDtypeStruct((batch, seq_len, d_model), x.dtype),
    )(x, emb_weight)
